# baseline (speedup 1.0000x reference)
import jax
import jax.numpy as jnp
from jax.experimental import pallas as pl
from jax.experimental.pallas import tpu as pltpu

_U = 8


def _sel_kernel(mask_ref, sel_ref, nsel_ref):
    c = mask_ref.shape[-1]

    def init(i, carry):
        sel_ref[0, i] = 0
        return carry

    jax.lax.fori_loop(0, c, init, 0)

    def body(i, cnt):
        nz = mask_ref[0, i] != 0.0

        @pl.when(nz)
        def _():
            sel_ref[0, cnt] = i

        return cnt + jnp.where(nz, 1, 0)

    cnt = jax.lax.fori_loop(0, c, body, 0)
    nsel_ref[0, 0] = cnt


def _copy_kernel(sel_ref, nsel_ref, *refs):
    del sel_ref
    ins = refs[:_U]
    out_ref = refs[_U]
    k = pl.program_id(0)
    nsel = nsel_ref[0]
    for u in range(_U):
        j = _U * k + u

        @pl.when(j < nsel)
        def _valid(u=u):
            out_ref[:, u : u + 1] = ins[u][...]

        @pl.when(j >= nsel)
        def _invalid(u=u):
            out_ref[:, u : u + 1] = jnp.full_like(ins[u], jnp.nan)


def kernel(input_tensor, indexes):
    n, c, h, w = input_tensor.shape

    sel, nsel = pl.pallas_call(
        _sel_kernel,
        in_specs=[pl.BlockSpec(memory_space=pltpu.MemorySpace.SMEM)],
        out_specs=(
            pl.BlockSpec(memory_space=pltpu.MemorySpace.SMEM),
            pl.BlockSpec(memory_space=pltpu.MemorySpace.SMEM),
        ),
        out_shape=(
            jax.ShapeDtypeStruct((1, c), jnp.int32),
            jax.ShapeDtypeStruct((1, 1), jnp.int32),
        ),
    )(indexes.reshape(1, c))
    sel = sel.reshape(c)
    nsel = nsel.reshape(1)

    def _in_spec(u):
        return pl.BlockSpec(
            (n, 1, h, w),
            lambda k, sel_ref, nsel_ref: (0, sel_ref[_U * k + u], 0, 0),
        )

    grid_spec = pltpu.PrefetchScalarGridSpec(
        num_scalar_prefetch=2,
        grid=(c // _U,),
        in_specs=[_in_spec(u) for u in range(_U)],
        out_specs=pl.BlockSpec(
            (n, _U, h, w), lambda k, sel_ref, nsel_ref: (0, k, 0, 0)
        ),
    )
    return pl.pallas_call(
        _copy_kernel,
        grid_spec=grid_spec,
        out_shape=jax.ShapeDtypeStruct((n, c, h, w), input_tensor.dtype),
        compiler_params=pltpu.CompilerParams(
            dimension_semantics=("parallel",),
            vmem_limit_bytes=64 * 1024 * 1024,
        ),
    )(sel, nsel, *([input_tensor] * _U))


# speedup vs baseline: 1.0012x; 1.0012x over previous
import jax
import jax.numpy as jnp
from jax.experimental import pallas as pl
from jax.experimental.pallas import tpu as pltpu

_U = 8


def _sel_kernel(mask_ref, sel_ref, nsel_ref):
    c = mask_ref.shape[-1]
    nz = mask_ref[...] != 0.0
    nzi = nz.astype(jnp.int32)
    row = jax.lax.broadcasted_iota(jnp.int32, (c, c), 0)
    col = jax.lax.broadcasted_iota(jnp.int32, (c, c), 1)
    rank = jnp.sum((nz & (col < row)).astype(jnp.int32), axis=1)
    m = nz & (jnp.broadcast_to(rank[None, :], (c, c)) == row)
    sel = jnp.sum(jnp.where(m, col, 0), axis=1)
    sel_ref[...] = sel.reshape(1, c)
    nsel_ref[...] = jnp.sum(nzi, axis=-1, keepdims=True)


def _copy_kernel(sel_ref, nsel_ref, *refs):
    del sel_ref
    ins = refs[:_U]
    out_ref = refs[_U]
    k = pl.program_id(0)
    nsel = nsel_ref[0]
    for u in range(_U):
        j = _U * k + u

        @pl.when(j < nsel)
        def _valid(u=u):
            out_ref[:, u : u + 1] = ins[u][...]

        @pl.when(j >= nsel)
        def _invalid(u=u):
            out_ref[:, u : u + 1] = jnp.full_like(ins[u], jnp.nan)


def kernel(input_tensor, indexes):
    n, c, h, w = input_tensor.shape

    sel, nsel = pl.pallas_call(
        _sel_kernel,
        out_shape=(
            jax.ShapeDtypeStruct((1, c), jnp.int32),
            jax.ShapeDtypeStruct((1, 1), jnp.int32),
        ),
    )(indexes.reshape(1, c))
    sel = sel.reshape(c)
    nsel = nsel.reshape(1)

    def _in_spec(u):
        return pl.BlockSpec(
            (n, 1, h, w),
            lambda k, sel_ref, nsel_ref: (0, sel_ref[_U * k + u], 0, 0),
        )

    grid_spec = pltpu.PrefetchScalarGridSpec(
        num_scalar_prefetch=2,
        grid=(c // _U,),
        in_specs=[_in_spec(u) for u in range(_U)],
        out_specs=pl.BlockSpec(
            (n, _U, h, w), lambda k, sel_ref, nsel_ref: (0, k, 0, 0)
        ),
    )
    return pl.pallas_call(
        _copy_kernel,
        grid_spec=grid_spec,
        out_shape=jax.ShapeDtypeStruct((n, c, h, w), input_tensor.dtype),
        compiler_params=pltpu.CompilerParams(
            dimension_semantics=("parallel",),
            vmem_limit_bytes=64 * 1024 * 1024,
        ),
    )(sel, nsel, *([input_tensor] * _U))
